# Initial kernel scaffold; baseline (speedup 1.0000x reference)
import jax
import jax.numpy as jnp
from jax.experimental import pallas as pl


def kernel(x, targets, table):
    raise NotImplementedError("write your pallas kernel here")


# speedup vs baseline: 1.6883x; 1.6883x over previous
import functools

import jax
import jax.numpy as jnp
from jax import lax
from jax.experimental import pallas as pl
from jax.experimental.pallas import tpu as pltpu
from jax.experimental.pallas import tpu_sc as plsc

V = 1000
C = 1000
N = 51200
NC, NS, L = 2, 16, 16
NW = NC * NS
BPW = N // NW
K = 32
ITERS = BPW // K
LSE_PAD = 1024


def _lse_body(table_ref, lse_ref):
    t = table_ref[...]
    m = jnp.max(t, axis=1)
    s = jnp.sum(jnp.exp(t - m[:, None]), axis=1)
    vals = jnp.log(s) + m
    lse_ref[...] = jnp.concatenate(
        [vals, jnp.zeros((LSE_PAD - V,), jnp.float32)])


def _loss_body(part_ref, out_ref):
    out_ref[...] = (jnp.sum(part_ref[...]) / N).reshape(1, 1)


def _sc_body(xf_hbm, tf_hbm, table_hbm, lse_hbm, out_hbm, part_hbm,
             xv0, xv1, tv0, tv1, rows0, rows1, lse_v, acc_v,
             gsem0, gsem1, osem0, osem1):
    wid = lax.axis_index("s") * NC + lax.axis_index("c")
    base0 = wid * BPW
    bufs = ((xv0, tv0, rows0, gsem0, osem0),
            (xv1, tv1, rows1, gsem1, osem1))

    pltpu.sync_copy(lse_hbm, lse_v)

    for b in range(2):
        xv, tv, rows, gsem, _ = bufs[b]
        pltpu.sync_copy(xf_hbm.at[pl.ds(base0 + b * K, K)], xv)
        pltpu.sync_copy(tf_hbm.at[pl.ds(base0 + b * K, K)], tv)
        pltpu.async_copy(table_hbm.at[xv], rows, gsem)

    iota = lax.broadcasted_iota(jnp.int32, (L,), 0)

    def step(i, acc):
        for b in range(2):
            xv, tv, rows, gsem, osem = bufs[b]
            it = 2 * i + b
            base = base0 + it * K
            pltpu.make_async_copy(table_hbm.at[xv], rows, gsem).wait()
            for g in range(K // L):
                rid = iota + g * L
                cid = tv[pl.ds(g * L, L)]
                xi = xv[pl.ds(g * L, L)]
                picked = plsc.load_gather(rows, [rid, cid])
                lses = plsc.load_gather(lse_v, [xi])
                acc = acc + (lses - picked)
            pltpu.async_copy(rows, out_hbm.at[pl.ds(base, K)], osem)

            @pl.when(i < ITERS // 2 - 1)
            def _():
                pltpu.make_async_copy(
                    rows, out_hbm.at[pl.ds(base, K)], osem).wait()
                nbase = base + 2 * K
                pltpu.sync_copy(xf_hbm.at[pl.ds(nbase, K)], xv)
                pltpu.sync_copy(tf_hbm.at[pl.ds(nbase, K)], tv)
                pltpu.async_copy(table_hbm.at[xv], rows, gsem)
        return acc

    acc = lax.fori_loop(0, ITERS // 2, step, jnp.zeros((L,), jnp.float32))

    for b in range(2):
        xv, tv, rows, _, osem = bufs[b]
        base = base0 + (ITERS - 2 + b) * K
        pltpu.make_async_copy(rows, out_hbm.at[pl.ds(base, K)], osem).wait()

    acc_v[...] = acc
    pltpu.sync_copy(acc_v, part_hbm.at[wid])


_sc_gather = pl.kernel(
    _sc_body,
    out_type=(
        jax.ShapeDtypeStruct((N, C), jnp.float32),
        jax.ShapeDtypeStruct((NW, L), jnp.float32),
    ),
    mesh=plsc.VectorSubcoreMesh(core_axis_name="c", subcore_axis_name="s"),
    compiler_params=pltpu.CompilerParams(
        use_tc_tiling_on_sc=False, needs_layout_passes=False),
    scratch_types=[
        pltpu.VMEM((K,), jnp.int32),
        pltpu.VMEM((K,), jnp.int32),
        pltpu.VMEM((K,), jnp.int32),
        pltpu.VMEM((K,), jnp.int32),
        pltpu.VMEM((K, C), jnp.float32),
        pltpu.VMEM((K, C), jnp.float32),
        pltpu.VMEM((LSE_PAD,), jnp.float32),
        pltpu.VMEM((L,), jnp.float32),
        pltpu.SemaphoreType.DMA,
        pltpu.SemaphoreType.DMA,
        pltpu.SemaphoreType.DMA,
        pltpu.SemaphoreType.DMA,
    ],
)

_lse_call = pl.pallas_call(
    _lse_body,
    out_shape=jax.ShapeDtypeStruct((LSE_PAD,), jnp.float32),
)

_loss_call = pl.pallas_call(
    _loss_body,
    out_shape=jax.ShapeDtypeStruct((1, 1), jnp.float32),
)


def kernel(x, targets, table):
    xf = x.reshape(-1).astype(jnp.int32)
    tf = targets.reshape(-1).astype(jnp.int32)
    table = table.astype(jnp.float32)
    lse = _lse_call(table)
    logits, partials = _sc_gather(xf, tf, table, lse)
    loss = _loss_call(partials)[0, 0]
    return (logits, loss)


# speedup vs baseline: 1.7140x; 1.0152x over previous
import functools

import jax
import jax.numpy as jnp
from jax import lax
from jax.experimental import pallas as pl
from jax.experimental.pallas import tpu as pltpu
from jax.experimental.pallas import tpu_sc as plsc

V = 1000
C = 1000
N = 51200
NC, NS, L = 2, 16, 16
NW = NC * NS
BPW = N // NW
K = 32
ITERS = BPW // K
LSE_PAD = 1024


def _lse_body(table_ref, lse_ref):
    t = table_ref[...]
    m = jnp.max(t, axis=1)
    s = jnp.sum(jnp.exp(t - m[:, None]), axis=1)
    vals = jnp.log(s) + m
    lse_ref[...] = jnp.concatenate(
        [vals, jnp.zeros((LSE_PAD - V,), jnp.float32)])


def _loss_body(part_ref, out_ref):
    out_ref[...] = (jnp.sum(part_ref[...]) / N).reshape(1, 1)


def _sc_body(xf_hbm, tf_hbm, table_hbm, lse_hbm, out_hbm, part_hbm,
             xa, ta, rows0, rows1, lse_v, acc_v,
             gsem0, gsem1, osem0, osem1):
    wid = lax.axis_index("s") * NC + lax.axis_index("c")
    base0 = wid * BPW
    bufs = ((rows0, gsem0, osem0), (rows1, gsem1, osem1))

    pltpu.sync_copy(xf_hbm.at[pl.ds(base0, BPW)], xa)
    pltpu.sync_copy(tf_hbm.at[pl.ds(base0, BPW)], ta)
    pltpu.sync_copy(lse_hbm, lse_v)

    for b in range(2):
        rows, gsem, _ = bufs[b]
        pltpu.async_copy(table_hbm.at[xa.at[pl.ds(b * K, K)]], rows, gsem)

    iota = lax.broadcasted_iota(jnp.int32, (L,), 0)

    def step(i, acc):
        for b in range(2):
            rows, gsem, osem = bufs[b]
            it = 2 * i + b
            off = it * K
            base = base0 + off
            pltpu.make_async_copy(
                table_hbm.at[xa.at[pl.ds(off, K)]], rows, gsem).wait()
            for g in range(K // L):
                rid = iota + g * L
                cid = ta[pl.ds(off + g * L, L)]
                xi = xa[pl.ds(off + g * L, L)]
                picked = plsc.load_gather(rows, [rid, cid])
                lses = plsc.load_gather(lse_v, [xi])
                acc = acc + (lses - picked)
            pltpu.async_copy(rows, out_hbm.at[pl.ds(base, K)], osem)

            @pl.when(i < ITERS // 2 - 1)
            def _():
                pltpu.make_async_copy(
                    rows, out_hbm.at[pl.ds(base, K)], osem).wait()
                pltpu.async_copy(
                    table_hbm.at[xa.at[pl.ds(off + 2 * K, K)]], rows, gsem)
        return acc

    acc = lax.fori_loop(0, ITERS // 2, step, jnp.zeros((L,), jnp.float32))

    for b in range(2):
        rows, _, osem = bufs[b]
        base = base0 + (ITERS - 2 + b) * K
        pltpu.make_async_copy(rows, out_hbm.at[pl.ds(base, K)], osem).wait()

    acc_v[...] = acc
    pltpu.sync_copy(acc_v, part_hbm.at[wid])


_sc_gather = pl.kernel(
    _sc_body,
    out_type=(
        jax.ShapeDtypeStruct((N, C), jnp.float32),
        jax.ShapeDtypeStruct((NW, L), jnp.float32),
    ),
    mesh=plsc.VectorSubcoreMesh(core_axis_name="c", subcore_axis_name="s"),
    compiler_params=pltpu.CompilerParams(
        use_tc_tiling_on_sc=False, needs_layout_passes=False),
    scratch_types=[
        pltpu.VMEM((BPW,), jnp.int32),
        pltpu.VMEM((BPW,), jnp.int32),
        pltpu.VMEM((K, C), jnp.float32),
        pltpu.VMEM((K, C), jnp.float32),
        pltpu.VMEM((LSE_PAD,), jnp.float32),
        pltpu.VMEM((L,), jnp.float32),
        pltpu.SemaphoreType.DMA,
        pltpu.SemaphoreType.DMA,
        pltpu.SemaphoreType.DMA,
        pltpu.SemaphoreType.DMA,
    ],
)

_lse_call = pl.pallas_call(
    _lse_body,
    out_shape=jax.ShapeDtypeStruct((LSE_PAD,), jnp.float32),
)

_loss_call = pl.pallas_call(
    _loss_body,
    out_shape=jax.ShapeDtypeStruct((1, 1), jnp.float32),
)


def kernel(x, targets, table):
    xf = x.reshape(-1).astype(jnp.int32)
    tf = targets.reshape(-1).astype(jnp.int32)
    table = table.astype(jnp.float32)
    lse = _lse_call(table)
    logits, partials = _sc_gather(xf, tf, table, lse)
    loss = _loss_call(partials)[0, 0]
    return (logits, loss)
